# Initial kernel scaffold; baseline (speedup 1.0000x reference)
import jax
import jax.numpy as jnp
from jax.experimental import pallas as pl


def kernel(in_feat, edge_index, edge_weight, graph_ids, anchor_embs, W, bias, prelu_a):
    raise NotImplementedError("write your pallas kernel here")


# speedup vs baseline: 7.4256x; 7.4256x over previous
import functools

import jax
import jax.numpy as jnp
from jax import lax
from jax.experimental import pallas as pl
from jax.experimental.pallas import tpu as pltpu
from jax.experimental.pallas import tpu_sc as plsc

_N, _E, _DIN, _DOUT, _G, _A = 10000, 320000, 128, 64, 256, 256

_NC, _NS = 2, 16
_NW = _NC * _NS
_EPW = _E // _NW
_CH = 80
_NCHUNK = _EPW // _CH
_NPAD = 10240
_NPT = _NPAD // _NS

_BN = 1000
_NBLK = _N // _BN


def _mm_body(x_ref, w_ref, o_ref):
    o_ref[...] = jnp.dot(x_ref[...], w_ref[...],
                         preferred_element_type=jnp.float32)


def _node_matmul(x, w):
    return pl.pallas_call(
        _mm_body,
        grid=(_NBLK,),
        in_specs=[pl.BlockSpec((_BN, _DIN), lambda i: (i, 0)),
                  pl.BlockSpec((_DIN, _DOUT), lambda i: (0, 0))],
        out_specs=pl.BlockSpec((_BN, _DOUT), lambda i: (i, 0)),
        out_shape=jax.ShapeDtypeStruct((_N, _DOUT), jnp.float32),
    )(x, w)


def _anchor_body(x_ref, w_ref, b_ref, a_ref, o_ref):
    hh = jnp.dot(x_ref[...], w_ref[...],
                 preferred_element_type=jnp.float32) + b_ref[...]
    o_ref[...] = jnp.maximum(hh, 0.0) + a_ref[...] * jnp.minimum(hh, 0.0)


def _anchor_path(x, w, b2, a2):
    return pl.pallas_call(
        _anchor_body,
        out_shape=jax.ShapeDtypeStruct((_A, _DOUT), jnp.float32),
    )(x, w, b2, a2)


def _sc_scatter(h, src2d, dst2d, w):
    mesh = plsc.VectorSubcoreMesh(core_axis_name="c", subcore_axis_name="s")

    @functools.partial(
        pl.kernel,
        mesh=mesh,
        out_type=jax.ShapeDtypeStruct((_NC * _NPAD, _DOUT), jnp.float32),
        compiler_params=pltpu.CompilerParams(use_tc_tiling_on_sc=False),
        scratch_types=[
            pltpu.VMEM((_NCHUNK, _CH), jnp.int32),
            pltpu.VMEM((_NCHUNK, _CH), jnp.int32),
            pltpu.VMEM((_EPW,), jnp.float32),
            pltpu.VMEM((_CH, _DOUT), jnp.float32),
            pltpu.VMEM((_NPT, _DOUT), jnp.float32),
            pltpu.VMEM_SHARED((_NPAD, _DOUT), jnp.float32),
            pltpu.SemaphoreType.DMA,
        ],
    )
    def sc_kernel(h_hbm, src_hbm, dst_hbm, w_hbm, out_hbm,
                  src_v, dst_v, w_v, rows_v, zero_v, agg_sh, sem):
        cid = lax.axis_index("c")
        sid = lax.axis_index("s")
        wid = sid * _NC + cid

        z16 = jnp.zeros((16,), jnp.float32)

        def _zero_row(r, c):
            for q in range(_DOUT // 16):
                zero_v[r, pl.ds(q * 16, 16)] = z16
            return c

        lax.fori_loop(0, _NPT, _zero_row, 0)
        pltpu.sync_copy(zero_v, agg_sh.at[pl.ds(sid * _NPT, _NPT)])

        pltpu.sync_copy(src_hbm.at[wid], src_v)
        pltpu.sync_copy(dst_hbm.at[wid], dst_v)
        pltpu.sync_copy(w_hbm.at[wid], w_v)

        plsc.subcore_barrier()

        def _chunk(i, c):
            pltpu.async_copy(h_hbm.at[src_v.at[i]], rows_v, sem).wait()
            for jb in range(_CH // 16):
                w16 = w_v[pl.ds(i * _CH + jb * 16, 16)]
                for t in range(16):
                    wj = lax.gather(
                        w16, jnp.full((16, 1), t, jnp.int32),
                        lax.GatherDimensionNumbers(
                            offset_dims=(), collapsed_slice_dims=(0,),
                            start_index_map=(0,)),
                        slice_sizes=(1,),
                        mode=lax.GatherScatterMode.PROMISE_IN_BOUNDS)
                    j = jb * 16 + t
                    for q in range(_DOUT // 16):
                        sl = pl.ds(q * 16, 16)
                        rows_v[j, sl] = rows_v[j, sl] * wj
            pltpu.sync_copy(rows_v, agg_sh.at[dst_v.at[i]], add=True)
            return c

        lax.fori_loop(0, _NCHUNK, _chunk, 0)

        plsc.subcore_barrier()
        pltpu.sync_copy(
            agg_sh.at[pl.ds(sid * _NPT, _NPT)],
            out_hbm.at[pl.ds(cid * _NPAD + sid * _NPT, _NPT)])

    return sc_kernel(h, src2d, dst2d, w)


def _pool_body(gid_ref, p0_ref, p1_ref, b_ref, a_ref,
               hout_ref, pool_ref, acc_ref, cnt_ref):
    i = pl.program_id(0)
    x = p0_ref[...] + p1_ref[...] + b_ref[...]
    h = jnp.maximum(x, 0.0) + a_ref[...] * jnp.minimum(x, 0.0)
    hout_ref[...] = h

    oh = (gid_ref[0] == lax.broadcasted_iota(jnp.int32, (_G, _BN), 0)
          ).astype(jnp.float32)

    @pl.when(i == 0)
    def _():
        acc_ref[...] = jnp.zeros_like(acc_ref)
        cnt_ref[...] = jnp.zeros_like(cnt_ref)

    acc_ref[...] += jnp.dot(oh, h, preferred_element_type=jnp.float32)
    cnt_ref[...] += jnp.sum(oh, axis=1, keepdims=True)

    @pl.when(i == _NBLK - 1)
    def _():
        pool_ref[...] = acc_ref[...] / jnp.maximum(cnt_ref[...], 1.0)


def _combine_pool(gid3, p0, p1, b2, a2):
    return pl.pallas_call(
        _pool_body,
        grid=(_NBLK,),
        in_specs=[
            pl.BlockSpec((1, 1, _BN), lambda i: (i, 0, 0)),
            pl.BlockSpec((_BN, _DOUT), lambda i: (i, 0)),
            pl.BlockSpec((_BN, _DOUT), lambda i: (i, 0)),
            pl.BlockSpec((1, _DOUT), lambda i: (0, 0)),
            pl.BlockSpec((1, 1), lambda i: (0, 0)),
        ],
        out_specs=[
            pl.BlockSpec((_BN, _DOUT), lambda i: (i, 0)),
            pl.BlockSpec((_G, _DOUT), lambda i: (0, 0)),
        ],
        out_shape=[jax.ShapeDtypeStruct((_N, _DOUT), jnp.float32),
                   jax.ShapeDtypeStruct((_G, _DOUT), jnp.float32)],
        scratch_shapes=[pltpu.VMEM((_G, _DOUT), jnp.float32),
                        pltpu.VMEM((_G, 1), jnp.float32)],
    )(gid3, p0, p1, b2, a2)


def kernel(in_feat, edge_index, edge_weight, graph_ids, anchor_embs,
           W, bias, prelu_a):
    b2 = bias.reshape(1, _DOUT)
    a2 = prelu_a.reshape(1, 1)
    h = _node_matmul(in_feat, W)
    anchor_out = _anchor_path(anchor_embs, W, b2, a2)
    src3d = edge_index[0].astype(jnp.int32).reshape(_NW, _NCHUNK, _CH)
    dst3d = edge_index[1].astype(jnp.int32).reshape(_NW, _NCHUNK, _CH)
    w2d = edge_weight.reshape(_NW, _EPW)
    partial = _sc_scatter(h, src3d, dst3d, w2d)
    gid3 = graph_ids.astype(jnp.int32).reshape(_NBLK, 1, _BN)
    h_out, pool = _combine_pool(gid3, partial[:_N], partial[_NPAD:_NPAD + _N],
                                b2, a2)
    return (h_out, pool, anchor_out)


# speedup vs baseline: 10.4106x; 1.4020x over previous
import functools

import jax
import jax.numpy as jnp
from jax import lax
from jax.experimental import pallas as pl
from jax.experimental.pallas import tpu as pltpu
from jax.experimental.pallas import tpu_sc as plsc

_N, _E, _DIN, _DOUT, _G, _A = 10000, 320000, 128, 64, 256, 256

_NC, _NS = 2, 16
_NW = _NC * _NS
_EPW = _E // _NW
_CH = 80
_NCHUNK = _EPW // _CH
_NPAD = 10240
_NPT = _NPAD // _NS

_BN = 1000
_NBLK = _N // _BN


def _mm_body(x_ref, w_ref, o_ref):
    o_ref[...] = jnp.dot(x_ref[...], w_ref[...],
                         preferred_element_type=jnp.float32)


def _node_matmul(x, w):
    return pl.pallas_call(
        _mm_body,
        grid=(_NBLK,),
        in_specs=[pl.BlockSpec((_BN, _DIN), lambda i: (i, 0)),
                  pl.BlockSpec((_DIN, _DOUT), lambda i: (0, 0))],
        out_specs=pl.BlockSpec((_BN, _DOUT), lambda i: (i, 0)),
        out_shape=jax.ShapeDtypeStruct((_N, _DOUT), jnp.float32),
    )(x, w)


def _anchor_body(x_ref, w_ref, b_ref, a_ref, o_ref):
    hh = jnp.dot(x_ref[...], w_ref[...],
                 preferred_element_type=jnp.float32) + b_ref[...]
    o_ref[...] = jnp.maximum(hh, 0.0) + a_ref[...] * jnp.minimum(hh, 0.0)


def _anchor_path(x, w, b2, a2):
    return pl.pallas_call(
        _anchor_body,
        out_shape=jax.ShapeDtypeStruct((_A, _DOUT), jnp.float32),
    )(x, w, b2, a2)


def _sc_scatter(h, src2d, dst2d, w):
    mesh = plsc.VectorSubcoreMesh(core_axis_name="c", subcore_axis_name="s")

    @functools.partial(
        pl.kernel,
        mesh=mesh,
        out_type=jax.ShapeDtypeStruct((_NC * _NPAD, _DOUT), jnp.float32),
        compiler_params=pltpu.CompilerParams(use_tc_tiling_on_sc=False),
        scratch_types=[
            pltpu.VMEM((_NCHUNK, _CH), jnp.int32),
            pltpu.VMEM((_NCHUNK, _CH), jnp.int32),
            pltpu.VMEM((_EPW,), jnp.float32),
            pltpu.VMEM((_CH, _DOUT), jnp.float32),
            pltpu.VMEM((_CH, _DOUT), jnp.float32),
            pltpu.VMEM((_NPT, _DOUT), jnp.float32),
            pltpu.VMEM_SHARED((_NPAD, _DOUT), jnp.float32),
            pltpu.SemaphoreType.DMA,
            pltpu.SemaphoreType.DMA,
            pltpu.SemaphoreType.DMA,
            pltpu.SemaphoreType.DMA,
        ],
    )
    def sc_kernel(h_hbm, src_hbm, dst_hbm, w_hbm, out_hbm,
                  src_v, dst_v, w_v, rows0, rows1, zero_v, agg_sh,
                  sem_g0, sem_g1, sem_s0, sem_s1):
        cid = lax.axis_index("c")
        sid = lax.axis_index("s")
        wid = sid * _NC + cid

        z16 = jnp.zeros((16,), jnp.float32)

        def _zero_row(r, c):
            for q in range(_DOUT // 16):
                zero_v[r, pl.ds(q * 16, 16)] = z16
            return c

        lax.fori_loop(0, _NPT, _zero_row, 0)
        pltpu.sync_copy(zero_v, agg_sh.at[pl.ds(sid * _NPT, _NPT)])

        pltpu.sync_copy(src_hbm.at[wid], src_v)
        pltpu.sync_copy(dst_hbm.at[wid], dst_v)
        pltpu.sync_copy(w_hbm.at[wid], w_v)

        plsc.subcore_barrier()

        def _scale(i, buf):
            for jb in range(_CH // 16):
                w16 = w_v[pl.ds(i * _CH + jb * 16, 16)]
                for t in range(16):
                    wj = lax.gather(
                        w16, jnp.full((16, 1), t, jnp.int32),
                        lax.GatherDimensionNumbers(
                            offset_dims=(), collapsed_slice_dims=(0,),
                            start_index_map=(0,)),
                        slice_sizes=(1,),
                        mode=lax.GatherScatterMode.PROMISE_IN_BOUNDS)
                    j = jb * 16 + t
                    for q in range(_DOUT // 16):
                        sl = pl.ds(q * 16, 16)
                        buf[j, sl] = buf[j, sl] * wj

        last = _NCHUNK - 1
        pltpu.async_copy(h_hbm.at[src_v.at[0]], rows0, sem_g0)
        pltpu.async_copy(h_hbm.at[src_v.at[1]], rows1, sem_g1)

        def _pair(k, c):
            i = k * 2
            pltpu.make_async_copy(h_hbm.at[src_v.at[i]], rows0, sem_g0).wait()
            _scale(i, rows0)
            pltpu.async_copy(rows0, agg_sh.at[dst_v.at[i]], sem_s0, add=True)
            pltpu.make_async_copy(
                h_hbm.at[src_v.at[i + 1]], rows1, sem_g1).wait()
            _scale(i + 1, rows1)
            pltpu.async_copy(
                rows1, agg_sh.at[dst_v.at[i + 1]], sem_s1, add=True)
            pltpu.make_async_copy(
                rows0, agg_sh.at[dst_v.at[i]], sem_s0).wait()
            pltpu.async_copy(
                h_hbm.at[src_v.at[jnp.minimum(i + 2, last)]], rows0, sem_g0)
            pltpu.make_async_copy(
                rows1, agg_sh.at[dst_v.at[i + 1]], sem_s1).wait()
            pltpu.async_copy(
                h_hbm.at[src_v.at[jnp.minimum(i + 3, last)]], rows1, sem_g1)
            return c

        lax.fori_loop(0, _NCHUNK // 2, _pair, 0)
        pltpu.make_async_copy(h_hbm.at[src_v.at[last]], rows0, sem_g0).wait()
        _scale(last, rows0)
        pltpu.sync_copy(rows0, agg_sh.at[dst_v.at[last]], add=True)
        pltpu.make_async_copy(h_hbm.at[src_v.at[last]], rows1, sem_g1).wait()

        plsc.subcore_barrier()
        pltpu.sync_copy(
            agg_sh.at[pl.ds(sid * _NPT, _NPT)],
            out_hbm.at[pl.ds(cid * _NPAD + sid * _NPT, _NPT)])

    return sc_kernel(h, src2d, dst2d, w)


def _pool_body(gid_ref, p0_ref, p1_ref, b_ref, a_ref,
               hout_ref, pool_ref, acc_ref, cnt_ref):
    i = pl.program_id(0)
    x = p0_ref[...] + p1_ref[...] + b_ref[...]
    h = jnp.maximum(x, 0.0) + a_ref[...] * jnp.minimum(x, 0.0)
    hout_ref[...] = h

    oh = (gid_ref[0] == lax.broadcasted_iota(jnp.int32, (_G, _BN), 0)
          ).astype(jnp.float32)

    @pl.when(i == 0)
    def _():
        acc_ref[...] = jnp.zeros_like(acc_ref)
        cnt_ref[...] = jnp.zeros_like(cnt_ref)

    acc_ref[...] += jnp.dot(oh, h, preferred_element_type=jnp.float32)
    cnt_ref[...] += jnp.sum(oh, axis=1, keepdims=True)

    @pl.when(i == _NBLK - 1)
    def _():
        pool_ref[...] = acc_ref[...] / jnp.maximum(cnt_ref[...], 1.0)


def _combine_pool(gid3, p0, p1, b2, a2):
    return pl.pallas_call(
        _pool_body,
        grid=(_NBLK,),
        in_specs=[
            pl.BlockSpec((1, 1, _BN), lambda i: (i, 0, 0)),
            pl.BlockSpec((_BN, _DOUT), lambda i: (i, 0)),
            pl.BlockSpec((_BN, _DOUT), lambda i: (i, 0)),
            pl.BlockSpec((1, _DOUT), lambda i: (0, 0)),
            pl.BlockSpec((1, 1), lambda i: (0, 0)),
        ],
        out_specs=[
            pl.BlockSpec((_BN, _DOUT), lambda i: (i, 0)),
            pl.BlockSpec((_G, _DOUT), lambda i: (0, 0)),
        ],
        out_shape=[jax.ShapeDtypeStruct((_N, _DOUT), jnp.float32),
                   jax.ShapeDtypeStruct((_G, _DOUT), jnp.float32)],
        scratch_shapes=[pltpu.VMEM((_G, _DOUT), jnp.float32),
                        pltpu.VMEM((_G, 1), jnp.float32)],
    )(gid3, p0, p1, b2, a2)


def kernel(in_feat, edge_index, edge_weight, graph_ids, anchor_embs,
           W, bias, prelu_a):
    b2 = bias.reshape(1, _DOUT)
    a2 = prelu_a.reshape(1, 1)
    h = _node_matmul(in_feat, W)
    anchor_out = _anchor_path(anchor_embs, W, b2, a2)
    src3d = edge_index[0].astype(jnp.int32).reshape(_NW, _NCHUNK, _CH)
    dst3d = edge_index[1].astype(jnp.int32).reshape(_NW, _NCHUNK, _CH)
    w2d = edge_weight.reshape(_NW, _EPW)
    partial = _sc_scatter(h, src3d, dst3d, w2d)
    gid3 = graph_ids.astype(jnp.int32).reshape(_NBLK, 1, _BN)
    h_out, pool = _combine_pool(gid3, partial[:_N], partial[_NPAD:_NPAD + _N],
                                b2, a2)
    return (h_out, pool, anchor_out)
